# baseline (speedup 1.0000x reference)
import functools

import jax
import jax.numpy as jnp
from jax import lax
from jax.experimental import pallas as pl

B, N, F, D, K = 8, 576, 192, 256, 10
_BIG = 1e30
_L = 128
_NP = 640
_IPS = 1


def _dot(a, b):
    return lax.dot_general(a, b, (((1,), (0,)), ((), ())),
                           preferred_element_type=jnp.float32)


def _dot_t(a, b):
    return lax.dot_general(a, b, (((1,), (1,)), ((), ())),
                           preferred_element_type=jnp.float32)


def _fused_body(xi_ref, wimg_ref, bimg_ref, wg_ref, bg_ref, we_ref, be_ref,
                xie_ref, gep_ref, out_ref):
    for s in range(_IPS):
        _one_image(s, xi_ref, wimg_ref, bimg_ref, wg_ref, bg_ref, we_ref,
                   be_ref, xie_ref, gep_ref, out_ref)


def _one_image(s, xi_ref, wimg_ref, bimg_ref, wg_ref, bg_ref, we_ref, be_ref,
               xie_ref, gep_ref, out_ref):
    x = xi_ref[s]
    xie = _dot(x, wimg_ref[...]) + bimg_ref[...]
    xie_ref[s] = xie

    sq = jnp.sum(xie * xie, axis=1, keepdims=True)
    gram = _dot_t(xie, xie)
    d2 = sq + jnp.reshape(sq, (1, N)) - 2.0 * gram

    row = lax.broadcasted_iota(jnp.int32, (N, N), 0)
    col = lax.broadcasted_iota(jnp.int32, (N, N), 1)
    d2 = jnp.where(row == col, d2 + 1e9, d2)
    d2p = jnp.concatenate(
        [d2, jnp.full((N, _NP - N), _BIG, jnp.float32)], axis=1)

    lanef = lax.broadcasted_iota(jnp.int32, (N, _L), 1).astype(jnp.float32)
    colf = lax.broadcasted_iota(jnp.int32, (N, _NP), 1).astype(jnp.float32)

    cur = d2p
    for p in range(K):
        mv = cur[:, 0:_L]
        mc = jnp.zeros((N, _L), jnp.float32)
        for c in range(1, 5):
            vc = cur[:, c * _L:(c + 1) * _L]
            take = vc < mv
            mv = jnp.where(take, vc, mv)
            mc = jnp.where(take, jnp.float32(c), mc)
        rowmin = jnp.min(mv, axis=1, keepdims=True)
        candg = jnp.where(mv == rowmin, mc * jnp.float32(_L) + lanef,
                          jnp.float32(1e9))
        ming = jnp.min(candg, axis=1, keepdims=True)
        if p < K - 1:
            cur = jnp.where(colf == ming, _BIG, cur)

    adj = jnp.where(
        (d2p < rowmin) | ((d2p == rowmin) & (colf <= ming)), 1.0, 0.0
    ).astype(jnp.float32)

    agg = _dot(adj[:, 0:N], xie)
    h = xie + agg / jnp.float32(K)
    gep = jnp.maximum(_dot(h, wg_ref[...]) + bg_ref[...], 0.0)
    gep_ref[s] = gep

    sv = jnp.sum(gep * we_ref[...], axis=0, keepdims=True)
    i = pl.program_id(0)
    out_ref[pl.ds(i * _IPS + s, 1), :] = (
        jnp.sum(sv, axis=1, keepdims=True) + be_ref[...])


@functools.partial(jax.jit, static_argnames=("interpret",))
def _run(xi, W_img, b_img2, W_g, b_g2, W_e2, b_e2, interpret=False):
    grid = (B // _IPS,)
    xie, gep, out = pl.pallas_call(
        _fused_body,
        grid=grid,
        in_specs=[
            pl.BlockSpec((_IPS, N, F), lambda b: (b, 0, 0)),
            pl.BlockSpec((F, D), lambda b: (0, 0)),
            pl.BlockSpec((1, D), lambda b: (0, 0)),
            pl.BlockSpec((D, D), lambda b: (0, 0)),
            pl.BlockSpec((1, D), lambda b: (0, 0)),
            pl.BlockSpec((N, D), lambda b: (0, 0)),
            pl.BlockSpec((1, 1), lambda b: (0, 0)),
        ],
        out_specs=[
            pl.BlockSpec((_IPS, N, D), lambda b: (b, 0, 0)),
            pl.BlockSpec((_IPS, N, D), lambda b: (b, 0, 0)),
            pl.BlockSpec((B, 1), lambda b: (0, 0)),
        ],
        out_shape=[
            jax.ShapeDtypeStruct((B, N, D), jnp.float32),
            jax.ShapeDtypeStruct((B, N, D), jnp.float32),
            jax.ShapeDtypeStruct((B, 1), jnp.float32),
        ],
        interpret=interpret,
    )(xi, W_img, b_img2, W_g, b_g2, W_e2, b_e2)
    return xie, gep, out


def kernel(xi, W_img, b_img, W_g, b_g, W_e, b_e):
    b_img2 = jnp.reshape(b_img, (1, D))
    b_g2 = jnp.reshape(b_g, (1, D))
    W_e2 = jnp.reshape(W_e, (N, D))
    b_e2 = jnp.reshape(b_e, (1, 1))
    return _run(xi, W_img, b_img2, W_g, b_g2, W_e2, b_e2)


# speedup vs baseline: 1.0104x; 1.0104x over previous
import functools

import jax
import jax.numpy as jnp
from jax import lax
from jax.experimental import pallas as pl

B, N, F, D, K = 8, 576, 192, 256, 10
_BIG = 1e30
_L = 128
_NP = 640
_IPS = 2


def _dot(a, b):
    return lax.dot_general(a, b, (((1,), (0,)), ((), ())),
                           preferred_element_type=jnp.float32)


def _dot_t(a, b):
    return lax.dot_general(a, b, (((1,), (1,)), ((), ())),
                           preferred_element_type=jnp.float32)


def _fused_body(xi_ref, wimg_ref, bimg_ref, wg_ref, bg_ref, we_ref, be_ref,
                xie_ref, gep_ref, out_ref):
    for s in range(_IPS):
        _one_image(s, xi_ref, wimg_ref, bimg_ref, wg_ref, bg_ref, we_ref,
                   be_ref, xie_ref, gep_ref, out_ref)


def _one_image(s, xi_ref, wimg_ref, bimg_ref, wg_ref, bg_ref, we_ref, be_ref,
               xie_ref, gep_ref, out_ref):
    x = xi_ref[s]
    xie = _dot(x, wimg_ref[...]) + bimg_ref[...]
    xie_ref[s] = xie

    sq = jnp.sum(xie * xie, axis=1, keepdims=True)
    gram = _dot_t(xie, xie)
    d2 = sq + jnp.reshape(sq, (1, N)) - 2.0 * gram

    row = lax.broadcasted_iota(jnp.int32, (N, N), 0)
    col = lax.broadcasted_iota(jnp.int32, (N, N), 1)
    d2 = jnp.where(row == col, d2 + 1e9, d2)
    d2p = jnp.concatenate(
        [d2, jnp.full((N, _NP - N), _BIG, jnp.float32)], axis=1)

    lanef = lax.broadcasted_iota(jnp.int32, (N, _L), 1).astype(jnp.float32)
    colf = lax.broadcasted_iota(jnp.int32, (N, _NP), 1).astype(jnp.float32)

    cur = d2p
    for p in range(K):
        mv = cur[:, 0:_L]
        mc = jnp.zeros((N, _L), jnp.float32)
        for c in range(1, 5):
            vc = cur[:, c * _L:(c + 1) * _L]
            take = vc < mv
            mv = jnp.where(take, vc, mv)
            mc = jnp.where(take, jnp.float32(c), mc)
        rowmin = jnp.min(mv, axis=1, keepdims=True)
        candg = jnp.where(mv == rowmin, mc * jnp.float32(_L) + lanef,
                          jnp.float32(1e9))
        ming = jnp.min(candg, axis=1, keepdims=True)
        if p < K - 1:
            cur = jnp.where(colf == ming, _BIG, cur)

    adj = jnp.where(
        (d2p < rowmin) | ((d2p == rowmin) & (colf <= ming)), 1.0, 0.0
    ).astype(jnp.float32)

    agg = _dot(adj[:, 0:N], xie)
    h = xie + agg / jnp.float32(K)
    gep = jnp.maximum(_dot(h, wg_ref[...]) + bg_ref[...], 0.0)
    gep_ref[s] = gep

    sv = jnp.sum(gep * we_ref[...], axis=0, keepdims=True)
    i = pl.program_id(0)
    out_ref[pl.ds(i * _IPS + s, 1), :] = (
        jnp.sum(sv, axis=1, keepdims=True) + be_ref[...])


@functools.partial(jax.jit, static_argnames=("interpret",))
def _run(xi, W_img, b_img2, W_g, b_g2, W_e2, b_e2, interpret=False):
    grid = (B // _IPS,)
    xie, gep, out = pl.pallas_call(
        _fused_body,
        grid=grid,
        in_specs=[
            pl.BlockSpec((_IPS, N, F), lambda b: (b, 0, 0)),
            pl.BlockSpec((F, D), lambda b: (0, 0)),
            pl.BlockSpec((1, D), lambda b: (0, 0)),
            pl.BlockSpec((D, D), lambda b: (0, 0)),
            pl.BlockSpec((1, D), lambda b: (0, 0)),
            pl.BlockSpec((N, D), lambda b: (0, 0)),
            pl.BlockSpec((1, 1), lambda b: (0, 0)),
        ],
        out_specs=[
            pl.BlockSpec((_IPS, N, D), lambda b: (b, 0, 0)),
            pl.BlockSpec((_IPS, N, D), lambda b: (b, 0, 0)),
            pl.BlockSpec((B, 1), lambda b: (0, 0)),
        ],
        out_shape=[
            jax.ShapeDtypeStruct((B, N, D), jnp.float32),
            jax.ShapeDtypeStruct((B, N, D), jnp.float32),
            jax.ShapeDtypeStruct((B, 1), jnp.float32),
        ],
        interpret=interpret,
    )(xi, W_img, b_img2, W_g, b_g2, W_e2, b_e2)
    return xie, gep, out


def kernel(xi, W_img, b_img, W_g, b_g, W_e, b_e):
    b_img2 = jnp.reshape(b_img, (1, D))
    b_g2 = jnp.reshape(b_g, (1, D))
    W_e2 = jnp.reshape(W_e, (N, D))
    b_e2 = jnp.reshape(b_e, (1, 1))
    return _run(xi, W_img, b_img2, W_g, b_g2, W_e2, b_e2)
